# Initial kernel scaffold; baseline (speedup 1.0000x reference)
import jax
import jax.numpy as jnp
from jax.experimental import pallas as pl


def kernel(x, edge_index):
    raise NotImplementedError("write your pallas kernel here")


# speedup vs baseline: 11.5651x; 11.5651x over previous
import functools

import jax
import jax.numpy as jnp
from jax import lax
from jax.experimental import pallas as pl
from jax.experimental.pallas import tpu as pltpu
from jax.experimental.pallas import tpu_sc as plsc

N_NODES = 10000
D = 128
N_EDGES = 320000

NC = 2
NS = 16
NW = NC * NS

CHUNK = 128
EPT = N_EDGES // NW
CPW = 80
BLK = 16
NBLK = CPW // BLK
EPT_PAD = CPW * CHUNK
PAD = EPT_PAD - EPT

N_ACC = 10112
N_DUMMY = N_ACC - N_NODES
ZROWS = N_ACC // NS

_mesh = plsc.VectorSubcoreMesh(core_axis_name="c", subcore_axis_name="s")


@functools.partial(
    pl.kernel,
    out_type=jax.ShapeDtypeStruct((NC, N_ACC, D), jnp.float32),
    mesh=_mesh,
    scratch_types=[
        pltpu.VMEM((BLK, CHUNK), jnp.int32),
        pltpu.VMEM((BLK, CHUNK), jnp.int32),
        pltpu.VMEM((BLK, CHUNK), jnp.int32),
        pltpu.VMEM((BLK, CHUNK), jnp.int32),
        pltpu.VMEM((CHUNK, D), jnp.float32),
        pltpu.VMEM((CHUNK, D), jnp.float32),
        pltpu.VMEM_SHARED((N_ACC, D), jnp.float32),
        pltpu.SemaphoreType.DMA,
        pltpu.SemaphoreType.DMA,
        pltpu.SemaphoreType.DMA,
        pltpu.SemaphoreType.DMA,
    ],
)
def _mp_sc(x_hbm, src_hbm, dst_hbm, zeros_hbm, out_hbm,
           src_a, src_b, dst_a, dst_b, rows_a, rows_b, accum,
           sem_ia, sem_ib, sem_a, sem_b):
    c = lax.axis_index("c")
    s = lax.axis_index("s")

    sbufs, dbufs, isems = (src_a, src_b), (dst_a, dst_b), (sem_ia, sem_ib)

    pltpu.async_copy(src_hbm.at[c, s, 0], src_a, sem_ia)
    pltpu.async_copy(dst_hbm.at[c, s, 0], dst_a, sem_ia)
    pltpu.sync_copy(zeros_hbm.at[pl.ds(s * ZROWS, ZROWS)],
                    accum.at[pl.ds(s * ZROWS, ZROWS)])
    plsc.subcore_barrier()

    for b in range(NBLK):
        sv, dv, isem = sbufs[b % 2], dbufs[b % 2], isems[b % 2]
        pltpu.make_async_copy(src_hbm.at[c, s, b], sv, isem).wait()
        pltpu.make_async_copy(dst_hbm.at[c, s, b], dv, isem).wait()
        if b + 1 < NBLK:
            pltpu.async_copy(src_hbm.at[c, s, b + 1],
                             sbufs[(b + 1) % 2], isems[(b + 1) % 2])
            pltpu.async_copy(dst_hbm.at[c, s, b + 1],
                             dbufs[(b + 1) % 2], isems[(b + 1) % 2])

        pltpu.async_copy(x_hbm.at[sv.at[0]], rows_a, sem_a)
        pltpu.async_copy(x_hbm.at[sv.at[1]], rows_b, sem_b)

        def body(jj, carry, sv=sv, dv=dv):
            j = jj * 2
            pltpu.make_async_copy(x_hbm.at[sv.at[j]], rows_a, sem_a).wait()
            pltpu.sync_copy(rows_a, accum.at[dv.at[j]], add=True)
            pltpu.async_copy(x_hbm.at[sv.at[j + 2]], rows_a, sem_a)
            pltpu.make_async_copy(x_hbm.at[sv.at[j + 1]], rows_b, sem_b).wait()
            pltpu.sync_copy(rows_b, accum.at[dv.at[j + 1]], add=True)
            pltpu.async_copy(x_hbm.at[sv.at[j + 3]], rows_b, sem_b)
            return carry

        lax.fori_loop(0, BLK // 2 - 1, body, 0)

        j = BLK - 2
        pltpu.make_async_copy(x_hbm.at[sv.at[j]], rows_a, sem_a).wait()
        pltpu.sync_copy(rows_a, accum.at[dv.at[j]], add=True)
        pltpu.make_async_copy(x_hbm.at[sv.at[j + 1]], rows_b, sem_b).wait()
        pltpu.sync_copy(rows_b, accum.at[dv.at[j + 1]], add=True)

    plsc.subcore_barrier()
    pltpu.sync_copy(accum.at[pl.ds(s * ZROWS, ZROWS)],
                    out_hbm.at[c, pl.ds(s * ZROWS, ZROWS)])


def _combine_body(p_ref, o_ref):
    o_ref[...] = p_ref[0] + p_ref[1]


_combine = pl.pallas_call(
    _combine_body,
    grid=(16,),
    in_specs=[pl.BlockSpec((2, N_ACC // 16, D), lambda i: (0, i, 0))],
    out_specs=pl.BlockSpec((N_ACC // 16, D), lambda i: (i, 0)),
    out_shape=jax.ShapeDtypeStruct((N_ACC, D), jnp.float32),
)


def kernel(x, edge_index):
    ei = edge_index.astype(jnp.int32)
    src = ei[0].reshape(NW, EPT)
    dst = ei[1].reshape(NW, EPT)
    pad_src = jnp.broadcast_to(
        (jnp.arange(PAD, dtype=jnp.int32) * 41) % N_NODES, (NW, PAD))
    pad_dst = jnp.broadcast_to(
        N_NODES + (jnp.arange(PAD, dtype=jnp.int32) % N_DUMMY), (NW, PAD))
    srcp = jnp.concatenate([src, pad_src], axis=1).reshape(
        NC, NS, NBLK, BLK, CHUNK)
    dstp = jnp.concatenate([dst, pad_dst], axis=1).reshape(
        NC, NS, NBLK, BLK, CHUNK)
    zeros = jnp.zeros((N_ACC, D), jnp.float32)
    partials = _mp_sc(x, srcp, dstp, zeros)
    return _combine(partials)[:N_NODES]
